# Initial kernel scaffold; baseline (speedup 1.0000x reference)
import jax
import jax.numpy as jnp
from jax.experimental import pallas as pl


def kernel(x, w):
    raise NotImplementedError("write your pallas kernel here")


# speedup vs baseline: 1.3065x; 1.3065x over previous
import functools

import jax
import jax.numpy as jnp
from jax import lax
from jax.experimental import pallas as pl
from jax.experimental.pallas import tpu as pltpu
from jax.experimental.pallas import tpu_sc as plsc

D_MODEL = 32
NUM_WORKERS = 32
CHUNK = 128


def _emb_body(n_chunks, w_hbm, idx_hbm, out_hbm, idx_v, rows_v, gsem):
    cid = lax.axis_index("c")
    sid = lax.axis_index("s")
    wid = sid * 2 + cid
    base = wid * (n_chunks * CHUNK)
    pltpu.sync_copy(idx_hbm.at[wid], idx_v)

    def body(j, carry):
        pltpu.async_copy(w_hbm.at[idx_v.at[j]], rows_v, gsem).wait()
        pltpu.sync_copy(rows_v, out_hbm.at[pl.ds(base + j * CHUNK, CHUNK)])
        return carry

    lax.fori_loop(0, n_chunks, body, 0)


def kernel(x, w):
    batch, seq = x.shape
    n_total = batch * seq
    n_per_worker = n_total // NUM_WORKERS
    n_chunks = n_per_worker // CHUNK
    idx3 = x.reshape(NUM_WORKERS, n_chunks, CHUNK)

    mesh = plsc.VectorSubcoreMesh(core_axis_name="c", subcore_axis_name="s")
    emb = functools.partial(
        pl.kernel,
        out_type=jax.ShapeDtypeStruct((n_total, D_MODEL), jnp.float32),
        mesh=mesh,
        scratch_types=[
            pltpu.VMEM((n_chunks, CHUNK), jnp.int32),
            pltpu.VMEM((CHUNK, D_MODEL), jnp.float32),
            pltpu.SemaphoreType.DMA,
        ],
        compiler_params=pltpu.CompilerParams(use_tc_tiling_on_sc=False),
    )(functools.partial(_emb_body, n_chunks))

    out = emb(w, idx3)
    return out.reshape(batch, seq, D_MODEL)


# speedup vs baseline: 1.4520x; 1.1113x over previous
import functools

import jax
import jax.numpy as jnp
from jax import lax
from jax.experimental import pallas as pl
from jax.experimental.pallas import tpu as pltpu
from jax.experimental.pallas import tpu_sc as plsc

D_MODEL = 32
NUM_WORKERS = 32
CHUNK = 128


NBUF = 4
LOOKAHEAD = 2


def _emb_body(n_chunks, w_hbm, idx_hbm, out_hbm, idx_v, rows_v, gsem, osem):
    cid = lax.axis_index("c")
    sid = lax.axis_index("s")
    wid = sid * 2 + cid
    base = wid * (n_chunks * CHUNK)
    pltpu.sync_copy(idx_hbm.at[wid], idx_v)

    def fire_gather(j):
        pltpu.async_copy(w_hbm.at[idx_v.at[j]], rows_v.at[lax.rem(j, NBUF)], gsem)

    def wait_gather(j):
        pltpu.make_async_copy(
            w_hbm.at[idx_v.at[j]], rows_v.at[lax.rem(j, NBUF)], gsem
        ).wait()

    def fire_store(j):
        pltpu.async_copy(
            rows_v.at[lax.rem(j, NBUF)], out_hbm.at[pl.ds(base + j * CHUNK, CHUNK)], osem
        )

    def wait_store_unit():
        pltpu.make_async_copy(
            rows_v.at[0], out_hbm.at[pl.ds(base, CHUNK)], osem
        ).wait()

    for j in range(LOOKAHEAD):
        fire_gather(j)

    def body(j, carry):
        wait_gather(j)
        fire_store(j)
        jn = j + LOOKAHEAD

        @pl.when(jn < n_chunks)
        def _():
            @pl.when(jn >= NBUF)
            def _():
                wait_store_unit()

            fire_gather(jn)

        return carry

    lax.fori_loop(0, n_chunks, body, 0)
    for _ in range(NBUF):
        wait_store_unit()


def kernel(x, w):
    batch, seq = x.shape
    n_total = batch * seq
    n_per_worker = n_total // NUM_WORKERS
    n_chunks = n_per_worker // CHUNK
    idx3 = x.reshape(NUM_WORKERS, n_chunks, CHUNK)

    mesh = plsc.VectorSubcoreMesh(core_axis_name="c", subcore_axis_name="s")
    emb = functools.partial(
        pl.kernel,
        out_type=jax.ShapeDtypeStruct((n_total, D_MODEL), jnp.float32),
        mesh=mesh,
        scratch_types=[
            pltpu.VMEM((n_chunks, CHUNK), jnp.int32),
            pltpu.VMEM((NBUF, CHUNK, D_MODEL), jnp.float32),
            pltpu.SemaphoreType.DMA,
            pltpu.SemaphoreType.DMA,
        ],
        compiler_params=pltpu.CompilerParams(use_tc_tiling_on_sc=False),
    )(functools.partial(_emb_body, n_chunks))

    out = emb(w, idx3)
    return out.reshape(batch, seq, D_MODEL)


# speedup vs baseline: 1.5006x; 1.0335x over previous
import functools

import jax
import jax.numpy as jnp
from jax import lax
from jax.experimental import pallas as pl
from jax.experimental.pallas import tpu as pltpu
from jax.experimental.pallas import tpu_sc as plsc

D_MODEL = 32
NUM_WORKERS = 32
CHUNK = 128


NBUF = 16
LOOKAHEAD = 12


def _emb_body(n_chunks, w_hbm, idx_hbm, out_hbm, idx_v, rows_v, gsem, osem):
    cid = lax.axis_index("c")
    sid = lax.axis_index("s")
    wid = sid * 2 + cid
    base = wid * (n_chunks * CHUNK)
    pltpu.sync_copy(idx_hbm.at[wid], idx_v)

    def fire_gather(j):
        pltpu.async_copy(w_hbm.at[idx_v.at[j]], rows_v.at[lax.rem(j, NBUF)], gsem)

    def wait_gather(j):
        pltpu.make_async_copy(
            w_hbm.at[idx_v.at[j]], rows_v.at[lax.rem(j, NBUF)], gsem
        ).wait()

    def fire_store(j):
        pltpu.async_copy(
            rows_v.at[lax.rem(j, NBUF)], out_hbm.at[pl.ds(base + j * CHUNK, CHUNK)], osem
        )

    def wait_store_unit():
        pltpu.make_async_copy(
            rows_v.at[0], out_hbm.at[pl.ds(base, CHUNK)], osem
        ).wait()

    for j in range(LOOKAHEAD):
        fire_gather(j)

    def body(j, carry):
        wait_gather(j)
        fire_store(j)
        jn = j + LOOKAHEAD

        @pl.when(jn < n_chunks)
        def _():
            @pl.when(jn >= NBUF)
            def _():
                wait_store_unit()

            fire_gather(jn)

        return carry

    lax.fori_loop(0, n_chunks, body, 0)
    for _ in range(NBUF):
        wait_store_unit()


def kernel(x, w):
    batch, seq = x.shape
    n_total = batch * seq
    n_per_worker = n_total // NUM_WORKERS
    n_chunks = n_per_worker // CHUNK
    idx3 = x.reshape(NUM_WORKERS, n_chunks, CHUNK)

    mesh = plsc.VectorSubcoreMesh(core_axis_name="c", subcore_axis_name="s")
    emb = functools.partial(
        pl.kernel,
        out_type=jax.ShapeDtypeStruct((n_total, D_MODEL), jnp.float32),
        mesh=mesh,
        scratch_types=[
            pltpu.VMEM((n_chunks, CHUNK), jnp.int32),
            pltpu.VMEM((NBUF, CHUNK, D_MODEL), jnp.float32),
            pltpu.SemaphoreType.DMA,
            pltpu.SemaphoreType.DMA,
        ],
        compiler_params=pltpu.CompilerParams(use_tc_tiling_on_sc=False),
    )(functools.partial(_emb_body, n_chunks))

    out = emb(w, idx3)
    return out.reshape(batch, seq, D_MODEL)
